# Initial kernel scaffold; baseline (speedup 1.0000x reference)
import jax
import jax.numpy as jnp
from jax.experimental import pallas as pl


def kernel(x, edge_index, batch, params):
    raise NotImplementedError("write your pallas kernel here")


# speedup vs baseline: 2.8134x; 2.8134x over previous
import functools

import jax
import jax.numpy as jnp
from jax import lax
from jax.experimental import pallas as pl
from jax.experimental.pallas import tpu as pltpu
from jax.experimental.pallas import tpu_sc as plsc

N_NODES = 50000
RT = 400
GRID_R = N_NODES // RT
FT = 32
NROW = 50176
RPW = NROW // 16
CHUNK = 128
DSUB = 196
NSUB = RPW // DSUB
NUM_GRAPHS = 8


def _mm_plain(h, wt):
  din, dout = wt.shape

  def body(h_ref, w_ref, o_ref):
    o_ref[...] = jnp.dot(h_ref[...], w_ref[...],
                         preferred_element_type=jnp.float32)

  return pl.pallas_call(
      body,
      grid=(GRID_R,),
      in_specs=[
          pl.BlockSpec((RT, din), lambda i: (i, 0)),
          pl.BlockSpec((din, dout), lambda i: (0, 0)),
      ],
      out_specs=pl.BlockSpec((RT, dout), lambda i: (i, 0)),
      out_shape=jax.ShapeDtypeStruct((h.shape[0], dout), jnp.float32),
  )(h, wt)


def _stats(m, deg_r=None, mean8=None):
  d = m.shape[1]
  weighted = deg_r is not None
  centered = mean8 is not None

  def body(*refs):
    refs = list(refs)
    m_ref = refs.pop(0)
    deg_ref = refs.pop(0) if weighted else None
    mn_ref = refs.pop(0) if centered else None
    o_ref = refs.pop(0)
    x = m_ref[...]
    if weighted:
      dis = (1.0 / jnp.sqrt(jnp.maximum(deg_ref[0, 0, :], 1.0)))[:, None]
      x = x * dis
    if centered:
      xc = x - mn_ref[0, :][None, :]
      s0 = jnp.sum(xc * xc, axis=0)
    else:
      s0 = jnp.sum(x, axis=0)
    rows = lax.broadcasted_iota(jnp.int32, (8, d), 0)
    upd = jnp.where(rows == 0, s0[None, :], 0.0)

    @pl.when(pl.program_id(0) == 0)
    def _():
      o_ref[...] = jnp.zeros_like(o_ref)

    o_ref[...] += upd

  in_specs = [pl.BlockSpec((RT, d), lambda i: (i, 0))]
  args = [m]
  if weighted:
    in_specs.append(pl.BlockSpec((1, 1, RT), lambda i: (i, 0, 0)))
    args.append(deg_r)
  if centered:
    in_specs.append(pl.BlockSpec((8, d), lambda i: (0, 0)))
    args.append(mean8)
  return pl.pallas_call(
      body,
      grid=(GRID_R,),
      in_specs=in_specs,
      out_specs=pl.BlockSpec((8, d), lambda i: (0, 0)),
      out_shape=jax.ShapeDtypeStruct((8, d), jnp.float32),
  )(*args)


def _bn_affine(m, g, b, count, deg_r=None):
  sums = _stats(m, deg_r)
  mean = sums[0] / count
  m8 = _pad8(jnp.stack([mean, jnp.zeros_like(mean)], axis=0))
  css = _stats(m, deg_r, mean8=m8)
  v = css[0] / count
  s = g / jnp.sqrt(v + 1e-5)
  t = b - mean * s
  return _pad8(jnp.stack([s, t], axis=0))


def _affine_relu_mm(p, st, wt):
  din, dout = wt.shape

  def body(p_ref, st_ref, w_ref, o_ref):
    s = st_ref[0, :][None, :]
    t = st_ref[1, :][None, :]
    h1 = jnp.maximum(p_ref[...] * s + t, 0.0)
    o_ref[...] = jnp.dot(h1, w_ref[...], preferred_element_type=jnp.float32)

  return pl.pallas_call(
      body,
      grid=(GRID_R,),
      in_specs=[
          pl.BlockSpec((RT, din), lambda i: (i, 0)),
          pl.BlockSpec((8, din), lambda i: (0, 0)),
          pl.BlockSpec((din, dout), lambda i: (0, 0)),
      ],
      out_specs=pl.BlockSpec((RT, dout), lambda i: (i, 0)),
      out_shape=jax.ShapeDtypeStruct((p.shape[0], dout), jnp.float32),
  )(p, st, wt)


def _row_scale(q, deg_r):
  d = q.shape[1]

  def body(q_ref, deg_ref, o_ref):
    dis = (1.0 / jnp.sqrt(jnp.maximum(deg_ref[0, 0, :], 1.0)))[:, None]
    o_ref[...] = q_ref[...] * dis

  return pl.pallas_call(
      body,
      grid=(GRID_R,),
      in_specs=[
          pl.BlockSpec((RT, d), lambda i: (i, 0)),
          pl.BlockSpec((1, 1, RT), lambda i: (i, 0, 0)),
      ],
      out_specs=pl.BlockSpec((RT, d), lambda i: (i, 0)),
      out_shape=jax.ShapeDtypeStruct(q.shape, jnp.float32),
  )(q, deg_r)


def _scale_affine_relu(s_raw, st, deg_r):
  d = s_raw.shape[1]

  def body(m_ref, st_ref, deg_ref, o_ref):
    s = st_ref[0, :][None, :]
    t = st_ref[1, :][None, :]
    dis = (1.0 / jnp.sqrt(jnp.maximum(deg_ref[0, 0, :], 1.0)))[:, None]
    o_ref[...] = jnp.maximum(m_ref[...] * dis * s + t, 0.0)

  return pl.pallas_call(
      body,
      grid=(GRID_R,),
      in_specs=[
          pl.BlockSpec((RT, d), lambda i: (i, 0)),
          pl.BlockSpec((8, d), lambda i: (0, 0)),
          pl.BlockSpec((1, 1, RT), lambda i: (i, 0, 0)),
      ],
      out_specs=pl.BlockSpec((RT, d), lambda i: (i, 0)),
      out_shape=jax.ShapeDtypeStruct(s_raw.shape, jnp.float32),
  )(s_raw, st, deg_r)


def _pool(h, batch_r):
  d = h.shape[1]

  def body(h_ref, b_ref, s_ref, c_ref):
    b = b_ref[0, 0, :]
    h = h_ref[...]
    rows = lax.broadcasted_iota(jnp.int32, (NUM_GRAPHS, d), 0)
    rows_c = lax.broadcasted_iota(jnp.int32, (NUM_GRAPHS, 128), 0)
    upd = jnp.zeros((NUM_GRAPHS, d), jnp.float32)
    updc = jnp.zeros((NUM_GRAPHS, 128), jnp.float32)
    for g in range(NUM_GRAPHS):
      mg = (b == g).astype(jnp.float32)
      sg = jnp.sum(h * mg[:, None], axis=0)
      cg = jnp.sum(mg)
      upd = jnp.where(rows == g, sg[None, :], upd)
      updc = jnp.where(rows_c == g, cg, updc)

    @pl.when(pl.program_id(0) == 0)
    def _():
      s_ref[...] = jnp.zeros_like(s_ref)
      c_ref[...] = jnp.zeros_like(c_ref)

    s_ref[...] += upd
    c_ref[...] += updc

  return pl.pallas_call(
      body,
      grid=(GRID_R,),
      in_specs=[
          pl.BlockSpec((RT, d), lambda i: (i, 0)),
          pl.BlockSpec((1, 1, RT), lambda i: (i, 0, 0)),
      ],
      out_specs=[
          pl.BlockSpec((NUM_GRAPHS, d), lambda i: (0, 0)),
          pl.BlockSpec((NUM_GRAPHS, 128), lambda i: (0, 0)),
      ],
      out_shape=[
          jax.ShapeDtypeStruct((NUM_GRAPHS, d), jnp.float32),
          jax.ShapeDtypeStruct((NUM_GRAPHS, 128), jnp.float32),
      ],
  )(h, batch_r)


def _decoder_layer(a, wt, bias_r=None, bn=False, gb_r=None, cb=512):
  din, dout = wt.shape
  nb = dout // cb

  def body(*refs):
    if bn:
      a_ref, w_ref, g_ref, b_ref, o_ref = refs
    elif bias_r is not None:
      a_ref, w_ref, b_ref, o_ref = refs
    else:
      a_ref, w_ref, o_ref = refs
    o = jnp.dot(a_ref[...], w_ref[...], preferred_element_type=jnp.float32)
    if bn:
      m = jnp.mean(o, axis=0, keepdims=True)
      d = o - m
      v = jnp.mean(d * d, axis=0, keepdims=True)
      g = g_ref[0, 0, :][None, :]
      b = b_ref[0, 0, :][None, :]
      o = jnp.maximum(g * (o - m) / jnp.sqrt(v + 1e-5) + b, 0.0)
    elif bias_r is not None:
      o = o + b_ref[0, 0, :][None, :]
    o_ref[...] = o

  in_specs = [
      pl.BlockSpec((8, din), lambda j: (0, 0)),
      pl.BlockSpec((din, cb), lambda j: (0, j)),
  ]
  args = [a, wt]
  if bn:
    in_specs.append(pl.BlockSpec((1, 1, cb), lambda j: (j, 0, 0)))
    in_specs.append(pl.BlockSpec((1, 1, cb), lambda j: (j, 0, 0)))
    args += [gb_r[0], gb_r[1]]
  elif bias_r is not None:
    in_specs.append(pl.BlockSpec((1, 1, cb), lambda j: (j, 0, 0)))
    args.append(bias_r)
  return pl.pallas_call(
      body,
      grid=(nb,),
      in_specs=in_specs,
      out_specs=pl.BlockSpec((8, cb), lambda j: (0, j)),
      out_shape=jax.ShapeDtypeStruct((8, dout), jnp.float32),
  )(*args)


def _sc_degree(dst_pad, nchunk):
  mesh = plsc.VectorSubcoreMesh(core_axis_name="c", subcore_axis_name="s")

  @functools.partial(
      pl.kernel,
      mesh=mesh,
      out_type=jax.ShapeDtypeStruct((NROW,), jnp.float32),
      scratch_types=[
          pltpu.VMEM_SHARED((NROW,), jnp.float32),
          pltpu.VMEM((RPW,), jnp.float32),
          pltpu.VMEM((CHUNK,), jnp.float32),
          pltpu.VMEM((CHUNK,), jnp.int32),
      ],
  )
  def k(dst_hbm, deg_hbm, acc, zeros_v, ones_v, dst_v):
    core = lax.axis_index("c")
    sub = lax.axis_index("s")

    @pl.when(core == 0)
    def _():
      def initz(i, _):
        zeros_v[pl.ds(i * 16, 16)] = jnp.zeros((16,), jnp.float32)
        return 0
      lax.fori_loop(0, RPW // 16, initz, 0)

      def inito(i, _):
        ones_v[pl.ds(i * 16, 16)] = jnp.ones((16,), jnp.float32)
        return 0
      lax.fori_loop(0, CHUNK // 16, inito, 0)

      pltpu.sync_copy(zeros_v, acc.at[pl.ds(sub * RPW, RPW)])
      plsc.subcore_barrier()

      def edge_chunk(k_, _):
        c = sub + 16 * k_
        pltpu.sync_copy(dst_hbm.at[pl.ds(c * CHUNK, CHUNK)], dst_v)
        pltpu.sync_copy(ones_v, acc.at[dst_v], add=True)
        return 0
      lax.fori_loop(0, nchunk // 16, edge_chunk, 0)

      plsc.subcore_barrier()
      pltpu.sync_copy(acc.at[pl.ds(sub * RPW, RPW)], zeros_v)
      pltpu.sync_copy(zeros_v, deg_hbm.at[pl.ds(sub * RPW, RPW)])

  return k(dst_pad)


def _sc_segsum(qp2d, src_pad, dst_pad, t_tiles, nchunk):
  mesh = plsc.VectorSubcoreMesh(core_axis_name="c", subcore_axis_name="s")

  @functools.partial(
      pl.kernel,
      mesh=mesh,
      out_type=jax.ShapeDtypeStruct((t_tiles * NROW, FT), jnp.float32),
      compiler_params=pltpu.CompilerParams(use_tc_tiling_on_sc=False),
      scratch_types=[
          pltpu.VMEM_SHARED((NROW, FT), jnp.float32),
          pltpu.VMEM((DSUB, FT), jnp.float32),
          pltpu.VMEM((CHUNK, FT), jnp.float32),
          pltpu.VMEM((CHUNK,), jnp.int32),
          pltpu.VMEM((CHUNK,), jnp.int32),
          pltpu.VMEM((CHUNK,), jnp.int32),
          pltpu.SemaphoreType.DMA,
      ],
  )
  def k(q_hbm, src_hbm, dst_hbm, s_hbm, acc, stage_v, rows_v, src_v, dst_v,
        gidx_v, sem):
    core = lax.axis_index("c")
    sub = lax.axis_index("s")

    for tl in range(t_tiles // 2):
      t = 2 * tl + core
      base = t * NROW

      def initz(i, _):
        stage_v[i, pl.ds(0, 16)] = jnp.zeros((16,), jnp.float32)
        stage_v[i, pl.ds(16, 16)] = jnp.zeros((16,), jnp.float32)
        return 0
      lax.fori_loop(0, DSUB, initz, 0)

      def zsub(j, _):
        pltpu.sync_copy(stage_v, acc.at[pl.ds(sub * RPW + j * DSUB, DSUB)])
        return 0
      lax.fori_loop(0, NSUB, zsub, 0)
      plsc.subcore_barrier()

      def edge_chunk(k_, _):
        c = sub + 16 * k_
        pltpu.sync_copy(src_hbm.at[pl.ds(c * CHUNK, CHUNK)], src_v)
        pltpu.sync_copy(dst_hbm.at[pl.ds(c * CHUNK, CHUNK)], dst_v)

        def addbase(i, _):
          gidx_v[pl.ds(i * 16, 16)] = src_v[pl.ds(i * 16, 16)] + base
          return 0
        lax.fori_loop(0, CHUNK // 16, addbase, 0)

        pltpu.async_copy(q_hbm.at[gidx_v], rows_v, sem).wait()
        pltpu.sync_copy(rows_v, acc.at[dst_v], add=True)
        return 0
      lax.fori_loop(0, nchunk // 16, edge_chunk, 0)

      plsc.subcore_barrier()

      def dsub(j, _):
        off = sub * RPW + j * DSUB
        pltpu.sync_copy(acc.at[pl.ds(off, DSUB)], stage_v)
        pltpu.sync_copy(stage_v, s_hbm.at[pl.ds(base + off, DSUB)])
        return 0
      lax.fori_loop(0, NSUB, dsub, 0)
      plsc.subcore_barrier()

  return k(qp2d, src_pad, dst_pad)


ENC_DOUT = [64, 64, 128, 256, 512]


def _pad8(st):
  return jnp.concatenate([st, jnp.zeros((6, st.shape[1]), jnp.float32)], 0)


def kernel(x, edge_index, batch, params):
  n = x.shape[0]
  assert n == N_NODES
  e = edge_index.shape[1]

  loop = jnp.arange(n, dtype=jnp.int32)
  src = jnp.concatenate([edge_index[0], loop])
  dst = jnp.concatenate([edge_index[1], loop])
  e_tot = e + n
  nchunk = -(-e_tot // CHUNK)
  nchunk = -(-nchunk // 16) * 16
  e_pad = nchunk * CHUNK
  src = jnp.concatenate([src, jnp.zeros((e_pad - e_tot,), jnp.int32)])
  dst = jnp.concatenate(
      [dst, jnp.full((e_pad - e_tot,), N_NODES, jnp.int32)])

  deg = _sc_degree(dst, nchunk)[:n]
  deg_r = deg.reshape(GRID_R, 1, RT)
  batch_r = batch.astype(jnp.int32).reshape(GRID_R, 1, RT)

  emb = {}
  h = jnp.pad(x, ((0, 0), (0, 123)))
  for i, dout in enumerate(ENC_DOUT):
    p = params["enc%d" % i]
    din = h.shape[1]
    w1t = jnp.transpose(p["mlp_w"])
    if i == 0:
      w1t = jnp.pad(w1t, ((0, 123), (0, 0)))
    pre = _mm_plain(h, w1t)
    st1 = _bn_affine(pre, p["bn1_g"], p["bn1_b"], n)
    qp = _row_scale(_affine_relu_mm(pre, st1, jnp.transpose(p["gcn_w"])),
                    deg_r)

    t_tiles = dout // FT
    qp3 = jnp.transpose(qp.reshape(n, t_tiles, FT), (1, 0, 2))
    qp3 = jnp.pad(qp3, ((0, 0), (0, NROW - n), (0, 0)))
    s2d = _sc_segsum(qp3.reshape(t_tiles * NROW, FT), src, dst,
                     t_tiles, nchunk)
    s_raw = jnp.transpose(
        s2d.reshape(t_tiles, NROW, FT)[:, :n, :], (1, 0, 2)).reshape(n, dout)

    st2 = _bn_affine(s_raw, p["bn2_g"], p["bn2_b"], n, deg_r)
    h = _scale_affine_relu(s_raw, st2, deg_r)
    emb["encoder%d" % i] = h

  sums, cnts = _pool(h, batch_r)
  pooled = sums / jnp.maximum(cnts[:, :1], 1.0)
  emb["gb_pool"] = pooled

  h = pooled
  for name in ("decoder2", "decoder1"):
    p = params[name]
    dout = p["w"].shape[0]
    nb = dout // 512
    gb_r = (p["bn_g"].reshape(nb, 1, 512), p["bn_b"].reshape(nb, 1, 512))
    h = _decoder_layer(h, jnp.transpose(p["w"]), bn=True, gb_r=gb_r, cb=512)
    emb[name] = h
  p = params["decoder0"]
  dout = p["w"].shape[0]
  dpad = 3840
  w0t = jnp.pad(jnp.transpose(p["w"]), ((0, 0), (0, dpad - dout)))
  b0_r = jnp.pad(p["b"], (0, dpad - dout)).reshape(6, 1, 640)
  h = _decoder_layer(h, w0t, bias_r=b0_r, cb=640)[:, :dout]
  emb["decoder0"] = h
  return h, emb


# speedup vs baseline: 4.6792x; 1.6632x over previous
import functools

import jax
import jax.numpy as jnp
from jax import lax
from jax.experimental import pallas as pl
from jax.experimental.pallas import tpu as pltpu
from jax.experimental.pallas import tpu_sc as plsc

N_NODES = 50000
RT = 400
GRID_R = N_NODES // RT
FT = 32
NROW = 50176
RPW = NROW // 16
CHUNK = 128
DSUB = 196
NSUB = RPW // DSUB
NUM_GRAPHS = 8


def _mm_plain(h, wt):
  din, dout = wt.shape

  def body(h_ref, w_ref, o_ref):
    o_ref[...] = jnp.dot(h_ref[...], w_ref[...],
                         preferred_element_type=jnp.float32)

  return pl.pallas_call(
      body,
      grid=(GRID_R,),
      in_specs=[
          pl.BlockSpec((RT, din), lambda i: (i, 0)),
          pl.BlockSpec((din, dout), lambda i: (0, 0)),
      ],
      out_specs=pl.BlockSpec((RT, dout), lambda i: (i, 0)),
      out_shape=jax.ShapeDtypeStruct((h.shape[0], dout), jnp.float32),
  )(h, wt)


def _stats(m, deg_r=None, mean8=None):
  d = m.shape[1]
  weighted = deg_r is not None
  centered = mean8 is not None

  def body(*refs):
    refs = list(refs)
    m_ref = refs.pop(0)
    deg_ref = refs.pop(0) if weighted else None
    mn_ref = refs.pop(0) if centered else None
    o_ref = refs.pop(0)
    x = m_ref[...]
    if weighted:
      dis = (1.0 / jnp.sqrt(jnp.maximum(deg_ref[0, 0, :], 1.0)))[:, None]
      x = x * dis
    if centered:
      xc = x - mn_ref[0, :][None, :]
      s0 = jnp.sum(xc * xc, axis=0)
    else:
      s0 = jnp.sum(x, axis=0)
    rows = lax.broadcasted_iota(jnp.int32, (8, d), 0)
    upd = jnp.where(rows == 0, s0[None, :], 0.0)

    @pl.when(pl.program_id(0) == 0)
    def _():
      o_ref[...] = jnp.zeros_like(o_ref)

    o_ref[...] += upd

  in_specs = [pl.BlockSpec((RT, d), lambda i: (i, 0))]
  args = [m]
  if weighted:
    in_specs.append(pl.BlockSpec((1, 1, RT), lambda i: (i, 0, 0)))
    args.append(deg_r)
  if centered:
    in_specs.append(pl.BlockSpec((8, d), lambda i: (0, 0)))
    args.append(mean8)
  return pl.pallas_call(
      body,
      grid=(GRID_R,),
      in_specs=in_specs,
      out_specs=pl.BlockSpec((8, d), lambda i: (0, 0)),
      out_shape=jax.ShapeDtypeStruct((8, d), jnp.float32),
  )(*args)


def _bn_affine(m, g, b, count, deg_r=None):
  sums = _stats(m, deg_r)
  mean = sums[0] / count
  m8 = _pad8(jnp.stack([mean, jnp.zeros_like(mean)], axis=0))
  css = _stats(m, deg_r, mean8=m8)
  v = css[0] / count
  s = g / jnp.sqrt(v + 1e-5)
  t = b - mean * s
  return _pad8(jnp.stack([s, t], axis=0))


def _affine_relu_mm(p, st, wt):
  din, dout = wt.shape

  def body(p_ref, st_ref, w_ref, o_ref):
    s = st_ref[0, :][None, :]
    t = st_ref[1, :][None, :]
    h1 = jnp.maximum(p_ref[...] * s + t, 0.0)
    o_ref[...] = jnp.dot(h1, w_ref[...], preferred_element_type=jnp.float32)

  return pl.pallas_call(
      body,
      grid=(GRID_R,),
      in_specs=[
          pl.BlockSpec((RT, din), lambda i: (i, 0)),
          pl.BlockSpec((8, din), lambda i: (0, 0)),
          pl.BlockSpec((din, dout), lambda i: (0, 0)),
      ],
      out_specs=pl.BlockSpec((RT, dout), lambda i: (i, 0)),
      out_shape=jax.ShapeDtypeStruct((p.shape[0], dout), jnp.float32),
  )(p, st, wt)


def _row_scale(q, deg_r):
  d = q.shape[1]

  def body(q_ref, deg_ref, o_ref):
    dis = (1.0 / jnp.sqrt(jnp.maximum(deg_ref[0, 0, :], 1.0)))[:, None]
    o_ref[...] = q_ref[...] * dis

  return pl.pallas_call(
      body,
      grid=(GRID_R,),
      in_specs=[
          pl.BlockSpec((RT, d), lambda i: (i, 0)),
          pl.BlockSpec((1, 1, RT), lambda i: (i, 0, 0)),
      ],
      out_specs=pl.BlockSpec((RT, d), lambda i: (i, 0)),
      out_shape=jax.ShapeDtypeStruct(q.shape, jnp.float32),
  )(q, deg_r)


def _scale_affine_relu(s_raw, st, deg_r):
  d = s_raw.shape[1]

  def body(m_ref, st_ref, deg_ref, o_ref):
    s = st_ref[0, :][None, :]
    t = st_ref[1, :][None, :]
    dis = (1.0 / jnp.sqrt(jnp.maximum(deg_ref[0, 0, :], 1.0)))[:, None]
    o_ref[...] = jnp.maximum(m_ref[...] * dis * s + t, 0.0)

  return pl.pallas_call(
      body,
      grid=(GRID_R,),
      in_specs=[
          pl.BlockSpec((RT, d), lambda i: (i, 0)),
          pl.BlockSpec((8, d), lambda i: (0, 0)),
          pl.BlockSpec((1, 1, RT), lambda i: (i, 0, 0)),
      ],
      out_specs=pl.BlockSpec((RT, d), lambda i: (i, 0)),
      out_shape=jax.ShapeDtypeStruct(s_raw.shape, jnp.float32),
  )(s_raw, st, deg_r)


def _pool(h, batch_r):
  d = h.shape[1]

  def body(h_ref, b_ref, s_ref, c_ref):
    b = b_ref[0, 0, :]
    h = h_ref[...]
    rows = lax.broadcasted_iota(jnp.int32, (NUM_GRAPHS, d), 0)
    rows_c = lax.broadcasted_iota(jnp.int32, (NUM_GRAPHS, 128), 0)
    upd = jnp.zeros((NUM_GRAPHS, d), jnp.float32)
    updc = jnp.zeros((NUM_GRAPHS, 128), jnp.float32)
    for g in range(NUM_GRAPHS):
      mg = (b == g).astype(jnp.float32)
      sg = jnp.sum(h * mg[:, None], axis=0)
      cg = jnp.sum(mg)
      upd = jnp.where(rows == g, sg[None, :], upd)
      updc = jnp.where(rows_c == g, cg, updc)

    @pl.when(pl.program_id(0) == 0)
    def _():
      s_ref[...] = jnp.zeros_like(s_ref)
      c_ref[...] = jnp.zeros_like(c_ref)

    s_ref[...] += upd
    c_ref[...] += updc

  return pl.pallas_call(
      body,
      grid=(GRID_R,),
      in_specs=[
          pl.BlockSpec((RT, d), lambda i: (i, 0)),
          pl.BlockSpec((1, 1, RT), lambda i: (i, 0, 0)),
      ],
      out_specs=[
          pl.BlockSpec((NUM_GRAPHS, d), lambda i: (0, 0)),
          pl.BlockSpec((NUM_GRAPHS, 128), lambda i: (0, 0)),
      ],
      out_shape=[
          jax.ShapeDtypeStruct((NUM_GRAPHS, d), jnp.float32),
          jax.ShapeDtypeStruct((NUM_GRAPHS, 128), jnp.float32),
      ],
  )(h, batch_r)


def _decoder_layer(a, wt, bias_r=None, bn=False, gb_r=None, cb=512):
  din, dout = wt.shape
  nb = dout // cb

  def body(*refs):
    if bn:
      a_ref, w_ref, g_ref, b_ref, o_ref = refs
    elif bias_r is not None:
      a_ref, w_ref, b_ref, o_ref = refs
    else:
      a_ref, w_ref, o_ref = refs
    o = jnp.dot(a_ref[...], w_ref[...], preferred_element_type=jnp.float32)
    if bn:
      m = jnp.mean(o, axis=0, keepdims=True)
      d = o - m
      v = jnp.mean(d * d, axis=0, keepdims=True)
      g = g_ref[0, 0, :][None, :]
      b = b_ref[0, 0, :][None, :]
      o = jnp.maximum(g * (o - m) / jnp.sqrt(v + 1e-5) + b, 0.0)
    elif bias_r is not None:
      o = o + b_ref[0, 0, :][None, :]
    o_ref[...] = o

  in_specs = [
      pl.BlockSpec((8, din), lambda j: (0, 0)),
      pl.BlockSpec((din, cb), lambda j: (0, j)),
  ]
  args = [a, wt]
  if bn:
    in_specs.append(pl.BlockSpec((1, 1, cb), lambda j: (j, 0, 0)))
    in_specs.append(pl.BlockSpec((1, 1, cb), lambda j: (j, 0, 0)))
    args += [gb_r[0], gb_r[1]]
  elif bias_r is not None:
    in_specs.append(pl.BlockSpec((1, 1, cb), lambda j: (j, 0, 0)))
    args.append(bias_r)
  return pl.pallas_call(
      body,
      grid=(nb,),
      in_specs=in_specs,
      out_specs=pl.BlockSpec((8, cb), lambda j: (0, j)),
      out_shape=jax.ShapeDtypeStruct((8, dout), jnp.float32),
  )(*args)


def _sc_degree(dst_pad, nchunk):
  mesh = plsc.VectorSubcoreMesh(core_axis_name="c", subcore_axis_name="s")

  @functools.partial(
      pl.kernel,
      mesh=mesh,
      out_type=jax.ShapeDtypeStruct((NROW,), jnp.float32),
      scratch_types=[
          pltpu.VMEM_SHARED((NROW,), jnp.float32),
          pltpu.VMEM((RPW,), jnp.float32),
          pltpu.VMEM((CHUNK,), jnp.float32),
          pltpu.VMEM((CHUNK,), jnp.int32),
      ],
  )
  def k(dst_hbm, deg_hbm, acc, zeros_v, ones_v, dst_v):
    core = lax.axis_index("c")
    sub = lax.axis_index("s")

    @pl.when(core == 0)
    def _():
      def initz(i, _):
        zeros_v[pl.ds(i * 16, 16)] = jnp.zeros((16,), jnp.float32)
        return 0
      lax.fori_loop(0, RPW // 16, initz, 0)

      def inito(i, _):
        ones_v[pl.ds(i * 16, 16)] = jnp.ones((16,), jnp.float32)
        return 0
      lax.fori_loop(0, CHUNK // 16, inito, 0)

      pltpu.sync_copy(zeros_v, acc.at[pl.ds(sub * RPW, RPW)])
      plsc.subcore_barrier()

      def edge_chunk(k_, _):
        c = sub + 16 * k_
        pltpu.sync_copy(dst_hbm.at[pl.ds(c * CHUNK, CHUNK)], dst_v)
        pltpu.sync_copy(ones_v, acc.at[dst_v], add=True)
        return 0
      lax.fori_loop(0, nchunk // 16, edge_chunk, 0)

      plsc.subcore_barrier()
      pltpu.sync_copy(acc.at[pl.ds(sub * RPW, RPW)], zeros_v)
      pltpu.sync_copy(zeros_v, deg_hbm.at[pl.ds(sub * RPW, RPW)])

  return k(dst_pad)


NBUF = 4


def _sc_segsum(qp2d, sd_pad, t_tiles, nchunk):
  mesh = plsc.VectorSubcoreMesh(core_axis_name="c", subcore_axis_name="s")
  scratch = [
      pltpu.VMEM_SHARED((NROW, FT), jnp.float32),
      pltpu.VMEM((DSUB, FT), jnp.float32),
  ]
  for _ in range(NBUF):
    scratch += [
        pltpu.VMEM((CHUNK, FT), jnp.float32),
        pltpu.VMEM((2, CHUNK), jnp.int32),
        pltpu.VMEM((CHUNK,), jnp.int32),
        pltpu.SemaphoreType.DMA,
        pltpu.SemaphoreType.DMA,
    ]

  @functools.partial(
      pl.kernel,
      mesh=mesh,
      out_type=jax.ShapeDtypeStruct((t_tiles * NROW, FT), jnp.float32),
      compiler_params=pltpu.CompilerParams(use_tc_tiling_on_sc=False),
      scratch_types=scratch,
  )
  def k(q_hbm, sd_hbm, s_hbm, acc, stage_v, *bufs):
    rows = bufs[0::5]
    sdv = bufs[1::5]
    gidx = bufs[2::5]
    sem_g = bufs[3::5]
    sem_s = bufs[4::5]
    core = lax.axis_index("c")
    sub = lax.axis_index("s")
    nouter = nchunk // (16 * NBUF)

    for tl in range(t_tiles // 2):
      t = 2 * tl + core
      base = t * NROW

      def initz(i, _):
        stage_v[i, pl.ds(0, 16)] = jnp.zeros((16,), jnp.float32)
        stage_v[i, pl.ds(16, 16)] = jnp.zeros((16,), jnp.float32)
        return 0
      lax.fori_loop(0, DSUB, initz, 0)

      def zsub(j, _):
        pltpu.sync_copy(stage_v, acc.at[pl.ds(sub * RPW + j * DSUB, DSUB)])
        return 0
      lax.fori_loop(0, NSUB, zsub, 0)
      plsc.subcore_barrier()

      def outer(ko, _):
        for b in range(NBUF):
          @pl.when(ko > 0)
          def _wait_prev():
            pltpu.make_async_copy(rows[b], acc.at[sdv[b].at[1]],
                                  sem_s[b]).wait()
          c = sub + 16 * (ko * NBUF + b)
          pltpu.sync_copy(sd_hbm.at[c], sdv[b])

          def addbase(i, _, b=b):
            gidx[b][pl.ds(i * 16, 16)] = sdv[b][0, pl.ds(i * 16, 16)] + base
            return 0
          lax.fori_loop(0, CHUNK // 16, addbase, 0)
          pltpu.async_copy(q_hbm.at[gidx[b]], rows[b], sem_g[b])
        for b in range(NBUF):
          pltpu.make_async_copy(q_hbm.at[gidx[b]], rows[b], sem_g[b]).wait()
          pltpu.async_copy(rows[b], acc.at[sdv[b].at[1]], sem_s[b], add=True)
        return 0
      lax.fori_loop(0, nouter, outer, 0)
      for b in range(NBUF):
        pltpu.make_async_copy(rows[b], acc.at[sdv[b].at[1]], sem_s[b]).wait()
      plsc.subcore_barrier()

      def dsub(j, _):
        off = sub * RPW + j * DSUB
        pltpu.sync_copy(acc.at[pl.ds(off, DSUB)], stage_v)
        pltpu.sync_copy(stage_v, s_hbm.at[pl.ds(base + off, DSUB)])
        return 0
      lax.fori_loop(0, NSUB, dsub, 0)
      plsc.subcore_barrier()

  return k(qp2d, sd_pad)


ENC_DOUT = [64, 64, 128, 256, 512]


def _pad8(st):
  return jnp.concatenate([st, jnp.zeros((6, st.shape[1]), jnp.float32)], 0)


def kernel(x, edge_index, batch, params):
  n = x.shape[0]
  assert n == N_NODES
  e = edge_index.shape[1]

  loop = jnp.arange(n, dtype=jnp.int32)
  src = jnp.concatenate([edge_index[0], loop])
  dst = jnp.concatenate([edge_index[1], loop])
  e_tot = e + n
  nchunk = -(-e_tot // CHUNK)
  nchunk = -(-nchunk // (16 * NBUF)) * 16 * NBUF
  e_pad = nchunk * CHUNK
  src = jnp.concatenate([src, jnp.zeros((e_pad - e_tot,), jnp.int32)])
  dst = jnp.concatenate(
      [dst, jnp.full((e_pad - e_tot,), N_NODES, jnp.int32)])
  sd = jnp.stack([src.reshape(nchunk, CHUNK), dst.reshape(nchunk, CHUNK)],
                 axis=1)

  deg = _sc_degree(dst, nchunk)[:n]
  deg_r = deg.reshape(GRID_R, 1, RT)
  batch_r = batch.astype(jnp.int32).reshape(GRID_R, 1, RT)

  emb = {}
  h = jnp.pad(x, ((0, 0), (0, 123)))
  for i, dout in enumerate(ENC_DOUT):
    p = params["enc%d" % i]
    din = h.shape[1]
    w1t = jnp.transpose(p["mlp_w"])
    if i == 0:
      w1t = jnp.pad(w1t, ((0, 123), (0, 0)))
    pre = _mm_plain(h, w1t)
    st1 = _bn_affine(pre, p["bn1_g"], p["bn1_b"], n)
    qp = _row_scale(_affine_relu_mm(pre, st1, jnp.transpose(p["gcn_w"])),
                    deg_r)

    t_tiles = dout // FT
    qp3 = jnp.transpose(qp.reshape(n, t_tiles, FT), (1, 0, 2))
    qp3 = jnp.pad(qp3, ((0, 0), (0, NROW - n), (0, 0)))
    s2d = _sc_segsum(qp3.reshape(t_tiles * NROW, FT), sd, t_tiles, nchunk)
    s_raw = jnp.transpose(
        s2d.reshape(t_tiles, NROW, FT)[:, :n, :], (1, 0, 2)).reshape(n, dout)

    st2 = _bn_affine(s_raw, p["bn2_g"], p["bn2_b"], n, deg_r)
    h = _scale_affine_relu(s_raw, st2, deg_r)
    emb["encoder%d" % i] = h

  sums, cnts = _pool(h, batch_r)
  pooled = sums / jnp.maximum(cnts[:, :1], 1.0)
  emb["gb_pool"] = pooled

  h = pooled
  for name in ("decoder2", "decoder1"):
    p = params[name]
    dout = p["w"].shape[0]
    nb = dout // 512
    gb_r = (p["bn_g"].reshape(nb, 1, 512), p["bn_b"].reshape(nb, 1, 512))
    h = _decoder_layer(h, jnp.transpose(p["w"]), bn=True, gb_r=gb_r, cb=512)
    emb[name] = h
  p = params["decoder0"]
  dout = p["w"].shape[0]
  dpad = 3840
  w0t = jnp.pad(jnp.transpose(p["w"]), ((0, 0), (0, dpad - dout)))
  b0_r = jnp.pad(p["b"], (0, dpad - dout)).reshape(6, 1, 640)
  h = _decoder_layer(h, w0t, bias_r=b0_r, cb=640)[:, :dout]
  emb["decoder0"] = h
  return h, emb
